# Initial kernel scaffold; baseline (speedup 1.0000x reference)
import jax
import jax.numpy as jnp
from jax.experimental import pallas as pl


def kernel(monomial_ids, coef_table, exp_table, special_table):
    raise NotImplementedError("write your pallas kernel here")


# speedup vs baseline: 3.0038x; 3.0038x over previous
import functools

import jax
import jax.numpy as jnp
from jax import lax
from jax.experimental import pallas as pl
from jax.experimental.pallas import tpu as pltpu
from jax.experimental.pallas import tpu_sc as plsc

D = 128
NSLOT = 10
NV = 8
SHIFT = 21
T = 64


def _sc_embed(ids_flat, coef_table, exp_table, special_table, n_tokens):
    info = plsc.get_sparse_core_info()
    nc, ns = info.num_cores, info.num_subcores
    nw = nc * ns
    per_w = n_tokens // nw
    n_chunks = per_w // T

    mesh = plsc.VectorSubcoreMesh(core_axis_name="c", subcore_axis_name="s")

    @functools.partial(
        pl.kernel, mesh=mesh,
        out_type=jax.ShapeDtypeStruct((n_tokens, D), jnp.float32),
        scratch_types=[
            pltpu.VMEM((NSLOT, T), jnp.int32),
            pltpu.VMEM((NSLOT, T, D), jnp.float32),
            pltpu.VMEM((T, D), jnp.float32),
            pltpu.SemaphoreType.DMA,
        ],
    )
    def k(ids_hbm, coef_hbm, exp_hbm, special_hbm, out_hbm, idxs, rows, outb, sem):
        wid = lax.axis_index("s") * nc + lax.axis_index("c")
        w0 = wid * per_w

        def chunk_body(ci, carry):
            base = w0 + ci * T
            for j in range(NSLOT):
                pltpu.sync_copy(ids_hbm.at[pl.ds(j * n_tokens + base, T)],
                                idxs.at[j])
            for j in range(1, 1 + NV):
                off = (j - 1) * SHIFT
                for kk in range(T // 16):
                    sl = pl.ds(kk * 16, 16)
                    idxs[j, sl] = idxs[j, sl] + off
            descs = [pltpu.async_copy(coef_hbm.at[idxs.at[0]], rows.at[0], sem)]
            for j in range(1, 1 + NV):
                descs.append(
                    pltpu.async_copy(exp_hbm.at[idxs.at[j]], rows.at[j], sem))
            descs.append(
                pltpu.async_copy(special_hbm.at[idxs.at[NSLOT - 1]],
                                 rows.at[NSLOT - 1], sem))
            for dsc in descs:
                dsc.wait()

            def tok_body(t, c2):
                for cc in range(D // 16):
                    sl = pl.ds(cc * 16, 16)
                    v = rows[0, t, sl]
                    for j in range(1, NSLOT):
                        v = v + rows[j, t, sl]
                    outb[t, sl] = v
                return c2
            lax.fori_loop(0, T, tok_body, 0)

            pltpu.sync_copy(outb, out_hbm.at[pl.ds(base, T)])
            return carry

        lax.fori_loop(0, n_chunks, chunk_body, 0)

    return k(ids_flat, coef_table, exp_table, special_table)


def kernel(monomial_ids, coef_table, exp_table, special_table):
    b, s, _ = monomial_ids.shape
    n = b * s
    ids_flat = monomial_ids.reshape(n, NSLOT).T.reshape(-1).astype(jnp.int32)
    out = _sc_embed(ids_flat, coef_table, exp_table, special_table, n)
    return out.reshape(b, s, D)


# speedup vs baseline: 3.0301x; 1.0088x over previous
import functools

import jax
import jax.numpy as jnp
from jax import lax
from jax.experimental import pallas as pl
from jax.experimental.pallas import tpu as pltpu
from jax.experimental.pallas import tpu_sc as plsc

D = 128
NSLOT = 10
NV = 8
SHIFT = 21
T = 128


def _sc_embed(ids_flat, coef_table, exp_table, special_table, n_tokens):
    info = plsc.get_sparse_core_info()
    nc, ns = info.num_cores, info.num_subcores
    nw = nc * ns
    per_w = n_tokens // nw
    n_chunks = per_w // T

    mesh = plsc.VectorSubcoreMesh(core_axis_name="c", subcore_axis_name="s")

    @functools.partial(
        pl.kernel, mesh=mesh,
        out_type=jax.ShapeDtypeStruct((n_tokens, D), jnp.float32),
        scratch_types=[
            pltpu.VMEM((NSLOT, T), jnp.int32),
            pltpu.VMEM((T, D), jnp.float32),
            pltpu.SemaphoreType.DMA,
        ],
    )
    def k(ids_hbm, coef_hbm, exp_hbm, special_hbm, out_hbm, idxs, acc, sem):
        wid = lax.axis_index("s") * nc + lax.axis_index("c")
        w0 = wid * per_w

        def chunk_body(ci, carry):
            base = w0 + ci * T
            for j in range(NSLOT):
                pltpu.sync_copy(ids_hbm.at[pl.ds(j * n_tokens + base, T)],
                                idxs.at[j])
            for j in range(1, 1 + NV):
                off = (j - 1) * SHIFT
                for kk in range(T // 16):
                    sl = pl.ds(kk * 16, 16)
                    idxs[j, sl] = idxs[j, sl] + off
            pltpu.async_copy(coef_hbm.at[idxs.at[0]], acc, sem).wait()
            descs = []
            for j in range(1, 1 + NV):
                descs.append(
                    pltpu.async_copy(exp_hbm.at[idxs.at[j]], acc, sem,
                                     add=True))
            descs.append(
                pltpu.async_copy(special_hbm.at[idxs.at[NSLOT - 1]], acc, sem,
                                 add=True))
            for dsc in descs:
                dsc.wait()

            pltpu.sync_copy(acc, out_hbm.at[pl.ds(base, T)])
            return carry

        lax.fori_loop(0, n_chunks, chunk_body, 0)

    return k(ids_flat, coef_table, exp_table, special_table)


def kernel(monomial_ids, coef_table, exp_table, special_table):
    b, s, _ = monomial_ids.shape
    n = b * s
    ids_flat = monomial_ids.reshape(n, NSLOT).T.reshape(-1).astype(jnp.int32)
    out = _sc_embed(ids_flat, coef_table, exp_table, special_table, n)
    return out.reshape(b, s, D)
